# Initial kernel scaffold; baseline (speedup 1.0000x reference)
import jax
import jax.numpy as jnp
from jax.experimental import pallas as pl


def kernel(x, params):
    raise NotImplementedError("write your pallas kernel here")


# speedup vs baseline: 1.4837x; 1.4837x over previous
import functools

import jax
import jax.numpy as jnp
from jax.experimental import pallas as pl

_F32 = jnp.float32


def _act(v, kind):
    if kind == 'relu':
        return jnp.maximum(v, 0.0)
    if kind == 'leaky':
        return jnp.where(v > 0, v, 0.1 * v)
    return v


def _conv_call(planes, groups, H, W, Co, act='relu', res=None,
               pre_act_res=False, row_tiles=1):
    TH = H // row_tiles
    metas = [[(p, r, c) for (p, r, c, _) in g] for g in groups]
    wlist = [jnp.concatenate([w for (_, _, _, w) in g],
                             axis=0).astype(jnp.bfloat16) for g in groups]
    planes = [p.astype(jnp.bfloat16) for p in planes]
    nP = len(planes)
    nG = len(groups)

    def body(*refs):
        prefs = refs[:nP]
        wrefs = refs[nP:nP + nG]
        rref = refs[nP + nG] if res is not None else None
        oref = refs[-1]
        r0 = pl.program_id(0) * TH
        acc = None
        for g, gm in enumerate(metas):
            xs = [prefs[pi][pl.ds(r0 + ro, TH), co:co + W, :]
                  for (pi, ro, co) in gm]
            xc = jnp.concatenate(xs, axis=-1) if len(xs) > 1 else xs[0]
            k = xc.shape[-1]
            d = jnp.dot(xc.reshape(TH * W, k), wrefs[g][...],
                        preferred_element_type=_F32)
            acc = d if acc is None else acc + d
        if rref is not None:
            rv = rref[...].reshape(TH * W, Co)
            out = _act(acc, act) + rv if pre_act_res else _act(acc + rv, act)
        else:
            out = _act(acc, act)
        oref[...] = out.reshape(TH, W, Co)

    in_specs = [pl.BlockSpec(p.shape, lambda i: (0, 0, 0)) for p in planes]
    in_specs += [pl.BlockSpec(w.shape, lambda i: (0, 0)) for w in wlist]
    args = list(planes) + wlist
    if res is not None:
        in_specs.append(pl.BlockSpec((TH, W, Co), lambda i: (i, 0, 0)))
        args.append(res)
    return pl.pallas_call(
        body,
        grid=(row_tiles,),
        in_specs=in_specs,
        out_specs=pl.BlockSpec((TH, W, Co), lambda i: (i, 0, 0)),
        out_shape=jax.ShapeDtypeStruct((H, W, Co), _F32),
    )(*args)


def _parity_planes(x, pad_lo, pad_hi):
    xp = jnp.pad(x, ((pad_lo, pad_hi), (pad_lo, pad_hi), (0, 0)))
    Hp, Wp, C = xp.shape
    xp = jnp.pad(xp, ((0, Hp % 2), (0, Wp % 2), (0, 0)))
    xr = xp.reshape(xp.shape[0] // 2, 2, xp.shape[1] // 2, 2, C)
    return {(a, b): xr[:, a, :, b, :] for a in (0, 1) for b in (0, 1)}


def _taps3(w, plane_idx=0):
    return [(plane_idx, dy, dx, w[:, :, dy, dx].T)
            for dy in range(3) for dx in range(3)]


def _stem_pool(x0, wstem):
    xp = jnp.pad(x0, ((2, 35), (2, 3), (0, 0)))
    parts = []
    wparts = []
    for dy in range(7):
        for dx in range(7):
            parts.append(jax.lax.slice(xp, (dy, dx, 0),
                                       (dy + 2 * 271 + 1, dx + 2 * 255 + 1, 3),
                                       (2, 2, 1)))
            wparts.append(wstem[:, :, dy, dx].T)
    X = jnp.concatenate(parts, axis=-1).reshape(272 * 256, 147)
    X = X.astype(jnp.bfloat16)
    wcat = jnp.concatenate(wparts, axis=0).astype(jnp.bfloat16)

    def body(xa, xb, wref, oref):
        xt = jnp.concatenate([xa[...], xb[...]], axis=0)
        s = jnp.maximum(jnp.dot(xt, wref[...], preferred_element_type=_F32),
                        0.0)
        s = s.reshape(32, 256, 64)
        gi = jax.lax.broadcasted_iota(jnp.int32, (32, 256, 64), 0)
        s = jnp.where(gi + pl.program_id(0) * 16 < 256, s, 0.0)
        t = s.reshape(32, 128, 2, 64)
        ev = t[:, :, 0, :]
        od = t[:, :, 1, :]
        evs = jnp.concatenate([ev[:, 1:, :], jnp.zeros((32, 1, 64), _F32)],
                              axis=1)
        c = jnp.maximum(jnp.maximum(ev, od), evs)
        t1 = c[0:16].reshape(8, 2, 128, 64)
        t2 = c[2:18].reshape(8, 2, 128, 64)
        oref[...] = jnp.maximum(jnp.maximum(t1[:, 0, :, :], t1[:, 1, :, :]),
                                t2[:, 0, :, :])

    return pl.pallas_call(
        body,
        grid=(16,),
        in_specs=[pl.BlockSpec((4096, 147), lambda i: (i, 0)),
                  pl.BlockSpec((4096, 147), lambda i: (i + 1, 0)),
                  pl.BlockSpec(wcat.shape, lambda i: (0, 0))],
        out_specs=pl.BlockSpec((8, 128, 64), lambda i: (i, 0, 0)),
        out_shape=jax.ShapeDtypeStruct((128, 128, 64), _F32),
    )(X, X, wcat)


def _resblock(x, w1, w2, wd, stride, row_tiles=1):
    H, W, _ = x.shape
    Co = w1.shape[0]
    if stride == 1:
        xp = jnp.pad(x, ((1, 1), (1, 1), (0, 0)))
        h1 = _conv_call([xp], [_taps3(w1)], H, W, Co, act='relu',
                        row_tiles=row_tiles)
        h1p = jnp.pad(h1, ((1, 1), (1, 1), (0, 0)))
        return _conv_call([h1p], [_taps3(w2)], H, W, Co, act='relu', res=x,
                          row_tiles=row_tiles)
    P = _parity_planes(x, 0, 1)
    order = [(0, 0), (0, 1), (1, 0), (1, 1)]
    pidx = {k: i for i, k in enumerate(order)}
    planes = [P[k] for k in order]
    Ho, Wo = H // 2, W // 2
    t1 = [(pidx[(dy & 1, dx & 1)], dy >> 1, dx >> 1, w1[:, :, dy, dx].T)
          for dy in range(3) for dx in range(3)]
    h1 = _conv_call(planes, [t1], Ho, Wo, Co, act='relu',
                    row_tiles=row_tiles)
    h1p = jnp.pad(h1, ((1, 1), (1, 1), (0, 0)))
    return _conv_call([h1p, P[(0, 0)]],
                      [_taps3(w2), [(1, 0, 0, wd[:, :, 0, 0].T)]],
                      Ho, Wo, Co, act='relu', row_tiles=row_tiles)


def _postproc_call(hf, wp2d, bp2d):
    def body(href, wref, bref, bb_ref, sc_ref, cl_ref, kp_ref):
        pr = jnp.dot(href[...].astype(jnp.bfloat16),
                     wref[...].astype(jnp.bfloat16),
                     preferred_element_type=_F32)
        pr = pr + bref[...]
        obj = jax.nn.sigmoid(pr[:, 0:1])
        cl = pr[:, 1:21]
        m = jnp.max(cl, axis=1, keepdims=True)
        e = jnp.exp(cl - m)
        p = e / jnp.sum(e, axis=1, keepdims=True) * obj
        scores = jnp.max(p, axis=1, keepdims=True)
        io20 = jax.lax.broadcasted_iota(jnp.int32, (256, 20), 1)
        clsf = jnp.min(jnp.where(p == scores, io20, 10000), axis=1,
                       keepdims=True).astype(_F32)
        tx = pr[:, 21:22]
        ty = pr[:, 22:23]
        tw = pr[:, 23:24]
        th = pr[:, 24:25]
        rid = jax.lax.broadcasted_iota(jnp.int32, (256, 1), 0)
        gy = (rid // 16).astype(_F32)
        gx = rid.astype(_F32) - gy * 16.0
        cx = (jax.nn.sigmoid(tx) + gx) * 32.0
        cy = (jax.nn.sigmoid(ty) + gy) * 32.0
        hw = jnp.maximum(tw, 0.0) * 256.0
        hh = jnp.maximum(th, 0.0) * 256.0
        x1 = (cx - hw) / 512.0
        y1 = (cy - hh) / 512.0
        x2 = (cx + hw) / 512.0
        y2 = (cy + hh) / 512.0
        bx1 = jnp.clip(x1 * 512.0, 0.0, 511.0) / 512.0
        by1 = jnp.clip(y1 * 512.0, 0.0, 511.0) / 512.0
        bx2 = jnp.clip(x2 * 512.0, 0.0, 511.0) / 512.0
        by2 = jnp.clip(y2 * 512.0, 0.0, 511.0) / 512.0
        bb_ref[...] = jnp.concatenate([bx1, by1, bx2, by2], axis=1)
        sc_ref[...] = scores
        cl_ref[...] = clsf
        off = clsf * 16.0
        ox1, oy1, ox2, oy2 = x1 + off, y1 + off, x2 + off, y2 + off
        valid = (scores >= 0.01).astype(_F32)
        s_row = scores.reshape(1, 256)
        ii = jax.lax.broadcasted_iota(jnp.int32, (256, 256), 0)
        jj = jax.lax.broadcasted_iota(jnp.int32, (256, 256), 1)
        cmp = (s_row > scores) | ((s_row == scores) & (jj < ii))
        rank = jnp.sum(jnp.where(cmp, 1, 0), axis=1, keepdims=True)
        R = (rank == jj).astype(_F32)

        def srt(v):
            return jax.lax.dot_general(R, v, (((0,), (0,)), ((), ())),
                                       preferred_element_type=_F32,
                                       precision=jax.lax.Precision.HIGHEST)

        xs1, ys1, xs2, ys2 = srt(ox1), srt(oy1), srt(ox2), srt(oy2)
        vs = srt(valid)
        areas = (xs2 - xs1) * (ys2 - ys1)
        xx1 = jnp.maximum(xs1, xs1.reshape(1, 256))
        yy1 = jnp.maximum(ys1, ys1.reshape(1, 256))
        xx2 = jnp.minimum(xs2, xs2.reshape(1, 256))
        yy2 = jnp.minimum(ys2, ys2.reshape(1, 256))
        iw = jnp.maximum(1e-28, xx2 - xx1)
        ih = jnp.maximum(1e-28, yy2 - yy1)
        inter = iw * ih
        ovr = inter / (areas + areas.reshape(1, 256) - inter)
        lane = jax.lax.broadcasted_iota(jnp.int32, (1, 256), 1)
        keep0 = vs.reshape(1, 256)

        def loop(k, keep):
            mrow = jnp.sum(jnp.where(ii == k, ovr, 0.0), axis=0,
                           keepdims=True)
            keepk = jnp.sum(jnp.where(lane == k, keep, 0.0))
            sup = (mrow > 0.45) & (lane > k) & (keepk > 0.5)
            return jnp.where(sup, 0.0, keep)

        keeps = jax.lax.fori_loop(0, 256, loop, keep0)
        kp_ref[...] = jnp.dot(R, keeps.reshape(256, 1),
                              preferred_element_type=_F32,
                              precision=jax.lax.Precision.HIGHEST)

    outs = pl.pallas_call(
        body,
        grid=(1,),
        in_specs=[pl.BlockSpec(hf.shape, lambda i: (0, 0)),
                  pl.BlockSpec(wp2d.shape, lambda i: (0, 0)),
                  pl.BlockSpec(bp2d.shape, lambda i: (0, 0))],
        out_specs=[pl.BlockSpec((256, 4), lambda i: (0, 0)),
                   pl.BlockSpec((256, 1), lambda i: (0, 0)),
                   pl.BlockSpec((256, 1), lambda i: (0, 0)),
                   pl.BlockSpec((256, 1), lambda i: (0, 0))],
        out_shape=[jax.ShapeDtypeStruct((256, 4), _F32),
                   jax.ShapeDtypeStruct((256, 1), _F32),
                   jax.ShapeDtypeStruct((256, 1), _F32),
                   jax.ShapeDtypeStruct((256, 1), _F32)],
    )(hf, wp2d, bp2d)
    return outs


@jax.jit
def kernel(x, params):
    p = params
    x0 = x[0].transpose(1, 2, 0)
    h = _stem_pool(x0, p['stem'])
    h = _resblock(h, p['l1b1w1'], p['l1b1w2'], None, 1, row_tiles=8)
    h = _resblock(h, p['l1b2w1'], p['l1b2w2'], None, 1, row_tiles=8)
    h = _resblock(h, p['l2b1w1'], p['l2b1w2'], p['l2d'], 2, row_tiles=4)
    h = _resblock(h, p['l2b2w1'], p['l2b2w2'], None, 1, row_tiles=4)
    h = _resblock(h, p['l3b1w1'], p['l3b1w2'], p['l3d'], 2, row_tiles=4)
    h = _resblock(h, p['l3b2w1'], p['l3b2w2'], None, 1, row_tiles=4)
    h = _resblock(h, p['l4b1w1'], p['l4b1w2'], p['l4d'], 2, row_tiles=2)
    h = _resblock(h, p['l4b2w1'], p['l4b2w2'], None, 1, row_tiles=2)
    hc = _conv_call([h], [[(0, 0, 0, p['cs1'][:, :, 0, 0].T)]], 16, 16, 256,
                    act='leaky')
    hcp = jnp.pad(hc, ((1, 1), (1, 1), (0, 0)))
    h2 = _conv_call([hcp], [_taps3(p['cs2'])], 16, 16, 512, act='leaky')
    b1 = _conv_call([h2], [[(0, 0, 0, p['br1'][:, :, 0, 0].T)]], 16, 16, 256,
                    act='leaky')
    b1p = jnp.pad(b1, ((1, 1), (1, 1), (0, 0)))
    hf = _conv_call([b1p], [_taps3(p['br2'])], 16, 16, 512, act='leaky',
                    res=h2, pre_act_res=True)
    wp2d = p['wp'][:, :, 0, 0].T
    bp2d = p['bp'].reshape(1, 25)
    bb, sc, cl, kp = _postproc_call(hf.reshape(256, 512), wp2d, bp2d)
    return (bb, sc.reshape(256), cl.reshape(256).astype(jnp.int32),
            kp.reshape(256) > 0.5)
